# Initial kernel scaffold; baseline (speedup 1.0000x reference)
import jax
import jax.numpy as jnp
from jax.experimental import pallas as pl


def kernel(nodes, node_positions, edges, graph_sizes, embed, pos_enc, type_W, type_b, gru_kernel, gru_rec, gru_b_in, gru_b_rec, att_w, att_b, att_v):
    raise NotImplementedError("write your pallas kernel here")


# speedup vs baseline: 5.7248x; 5.7248x over previous
import functools

import jax
import jax.numpy as jnp
from jax import lax
from jax.experimental import pallas as pl
from jax.experimental.pallas import tpu as pltpu
from jax.experimental.pallas import tpu_sc as plsc

NN = 10000
NNP = 10240
NE = 160000
D = 128
HD = 64
NT = 2
NG = 100
NC, NS, LANES = 2, 16, 16

ECH = 79
NEP = NS * ECH * 128
TRASH = 20800
ACC_ROWS = 21504
TPR = ACC_ROWS // NS

_mesh = plsc.VectorSubcoreMesh(core_axis_name="c", subcore_axis_name="s",
                               num_cores=NC, num_subcores=NS)


def _zero_rows(zref, width):
    zv = jnp.zeros((LANES,), jnp.float32)

    def body(i, _):
        for j in range(width // LANES):
            zref[i, pl.ds(j * LANES, LANES)] = zv
        return 0

    lax.fori_loop(0, zref.shape[0], body, 0)


def _fill_ones(oref, width):
    ov = jnp.ones((LANES,), jnp.float32)

    def body(i, _):
        for j in range(width // LANES):
            oref[i, pl.ds(j * LANES, LANES)] = ov
        return 0

    lax.fori_loop(0, oref.shape[0], body, 0)


@functools.partial(
    pl.kernel,
    out_type=jax.ShapeDtypeStruct((NC, NNP, HD), jnp.float32),
    mesh=_mesh,
    compiler_params=pltpu.CompilerParams(use_tc_tiling_on_sc=False),
    scratch_types=[
        pltpu.VMEM((5, 128), jnp.int32),
        pltpu.VMEM((5, 128), jnp.int32),
        pltpu.VMEM((128, HD), jnp.float32),
        pltpu.SemaphoreType.DMA,
    ],
)
def _sc_embed(embed_r, posx_r, idxn, idxp, out, idxn_v, idxp_v, rows_v, sem):
    c = lax.axis_index("c")
    s = lax.axis_index("s")
    pltpu.sync_copy(idxn.at[c, s], idxn_v)
    pltpu.sync_copy(idxp.at[c, s], idxp_v)
    for j in range(5):
        pltpu.async_copy(embed_r.at[idxn_v.at[j]], rows_v, sem).wait()
        pltpu.async_copy(posx_r.at[idxp_v.at[j]], rows_v, sem, add=True).wait()
        pltpu.sync_copy(rows_v, out.at[c, pl.ds(s * 640 + j * 128, 128)])


@functools.partial(
    pl.kernel,
    out_type=jax.ShapeDtypeStruct((NC, ACC_ROWS, HD), jnp.float32),
    mesh=_mesh,
    compiler_params=pltpu.CompilerParams(use_tc_tiling_on_sc=False),
    scratch_types=[
        pltpu.VMEM((ECH, 128), jnp.int32),
        pltpu.VMEM((ECH, 128), jnp.int32),
        pltpu.VMEM((128, HD), jnp.float32),
        pltpu.VMEM((64, HD), jnp.float32),
        pltpu.VMEM_SHARED((ACC_ROWS, HD), jnp.float32),
        pltpu.SemaphoreType.DMA,
    ],
)
def _sc_edge(states_flat, src_idx, dst_idx, a_out, src_v, dst_v, rows_v,
             zero_v, acc, sem):
    c = lax.axis_index("c")
    s = lax.axis_index("s")
    _zero_rows(zero_v, HD)

    def zbody(k, _):
        pltpu.sync_copy(zero_v, acc.at[pl.ds(s * TPR + k * 64, 64)])
        return 0

    lax.fori_loop(0, TPR // 64, zbody, 0)
    plsc.subcore_barrier()

    pltpu.sync_copy(src_idx.at[c, s], src_v)
    pltpu.sync_copy(dst_idx.at[s], dst_v)

    def ebody(j, _):
        pltpu.async_copy(states_flat.at[src_v.at[j]], rows_v, sem).wait()
        pltpu.sync_copy(rows_v, acc.at[dst_v.at[j]], add=True)
        return 0

    lax.fori_loop(0, ECH, ebody, 0)

    plsc.subcore_barrier()
    pltpu.sync_copy(acc.at[pl.ds(s * TPR, TPR)],
                    a_out.at[c, pl.ds(s * TPR, TPR)])


CCH = 40
NEPC = NC * NS * CCH * 128


@functools.partial(
    pl.kernel,
    out_type=jax.ShapeDtypeStruct((NC, ACC_ROWS, LANES), jnp.float32),
    mesh=_mesh,
    compiler_params=pltpu.CompilerParams(use_tc_tiling_on_sc=False),
    scratch_types=[
        pltpu.VMEM((CCH, 128), jnp.int32),
        pltpu.VMEM((128, LANES), jnp.float32),
        pltpu.VMEM((64, LANES), jnp.float32),
        pltpu.VMEM_SHARED((ACC_ROWS, LANES), jnp.float32),
    ],
)
def _sc_counts(dst_idx, cnt_out, dst_v, ones_v, zero_v, cnt):
    c = lax.axis_index("c")
    s = lax.axis_index("s")
    _zero_rows(zero_v, LANES)
    _fill_ones(ones_v, LANES)

    def zbody(k, _):
        pltpu.sync_copy(zero_v, cnt.at[pl.ds(s * TPR + k * 64, 64)])
        return 0

    lax.fori_loop(0, TPR // 64, zbody, 0)
    plsc.subcore_barrier()

    pltpu.sync_copy(dst_idx.at[c, s], dst_v)

    def ebody(j, _):
        pltpu.sync_copy(ones_v, cnt.at[dst_v.at[j]], add=True)
        return 0

    lax.fori_loop(0, CCH, ebody, 0)

    plsc.subcore_barrier()
    pltpu.sync_copy(cnt.at[pl.ds(s * TPR, TPR)],
                    cnt_out.at[c, pl.ds(s * TPR, TPR)])


def _tc_gru_body(a00, a01, a10, a11, c0a, c0b, c1a, c1b, s0, s1, tw, tb, gk,
                 gr, bi, br, out):
    w0 = tw[0]
    w1 = tw[1]
    c0 = c0a[0] + c0b[0]
    c1 = c1a[0] + c1b[0]
    msgs = (jnp.dot(a00[0], w0[:HD, :], preferred_element_type=jnp.float32)
            + jnp.dot(a10[0], w0[HD:, :], preferred_element_type=jnp.float32)
            + jnp.dot(a01[0], w1[:HD, :], preferred_element_type=jnp.float32)
            + jnp.dot(a11[0], w1[HD:, :], preferred_element_type=jnp.float32)
            + c0[:, 0:1] * tb[0:1, :] + c1[:, 0:1] * tb[1:2, :])
    h = jnp.concatenate([s0[0], s1[0]], axis=1)
    xm = jnp.dot(msgs, gk[:], preferred_element_type=jnp.float32) + bi[0:1, :]
    hm = jnp.dot(h, gr[:], preferred_element_type=jnp.float32) + br[0:1, :]
    z = jax.nn.sigmoid(xm[:, :D] + hm[:, :D])
    r = jax.nn.sigmoid(xm[:, D:2 * D] + hm[:, D:2 * D])
    hh = jnp.tanh(xm[:, 2 * D:] + r * hm[:, 2 * D:])
    hn = z * h + (1.0 - z) * hh
    out[0] = hn[:, :HD]
    out[1] = hn[:, HD:]


_BR = 1280


def _tc_gru(a, cnt, states, type_W, type_b, gk, gr, bi, br):
    grid = (NNP // _BR,)
    return pl.pallas_call(
        _tc_gru_body,
        grid=grid,
        in_specs=[
            pl.BlockSpec((1, _BR, HD), lambda i: (0, i, 0)),
            pl.BlockSpec((1, _BR, HD), lambda i: (0, i + 8, 0)),
            pl.BlockSpec((1, _BR, HD), lambda i: (1, i, 0)),
            pl.BlockSpec((1, _BR, HD), lambda i: (1, i + 8, 0)),
            pl.BlockSpec((1, _BR, LANES), lambda i: (0, i, 0)),
            pl.BlockSpec((1, _BR, LANES), lambda i: (1, i, 0)),
            pl.BlockSpec((1, _BR, LANES), lambda i: (0, i + 8, 0)),
            pl.BlockSpec((1, _BR, LANES), lambda i: (1, i + 8, 0)),
            pl.BlockSpec((1, _BR, HD), lambda i: (0, i, 0)),
            pl.BlockSpec((1, _BR, HD), lambda i: (1, i, 0)),
            pl.BlockSpec((NT, D, D), lambda i: (0, 0, 0)),
            pl.BlockSpec((NT, D), lambda i: (0, 0)),
            pl.BlockSpec((D, 3 * D), lambda i: (0, 0)),
            pl.BlockSpec((D, 3 * D), lambda i: (0, 0)),
            pl.BlockSpec((1, 3 * D), lambda i: (0, 0)),
            pl.BlockSpec((1, 3 * D), lambda i: (0, 0)),
        ],
        out_specs=pl.BlockSpec((NC, _BR, HD), lambda i: (0, i, 0)),
        out_shape=jax.ShapeDtypeStruct((NC, NNP, HD), jnp.float32),
    )(a, a, a, a, cnt, cnt, cnt, cnt, states, states, type_W, type_b, gk, gr,
      bi, br)


def _tc_att_body(s_ref, seg_ref, aw_ref, ab_ref, av_ref, out_ref):
    h = jnp.concatenate([s_ref[0, :NN, :], s_ref[1, :NN, :]], axis=1)
    scores = jnp.dot(h, aw_ref[:], preferred_element_type=jnp.float32) \
        + ab_ref[0, 0]
    m = jnp.max(scores)
    e = jnp.exp(scores - m)
    seg = seg_ref[:]
    sums = jnp.dot(seg, e, preferred_element_type=jnp.float32)
    vals = jnp.tanh(jnp.dot(h, av_ref[:], preferred_element_type=jnp.float32))
    num = jnp.dot(seg, e * vals, preferred_element_type=jnp.float32)
    out_ref[:] = num / (sums + 1e-16)


def _tc_att(states, seg, att_w, att_b, att_v):
    return pl.pallas_call(
        _tc_att_body,
        out_shape=jax.ShapeDtypeStruct((NG, D), jnp.float32),
    )(states, seg, att_w, att_b, att_v)


def kernel(nodes, node_positions, edges, graph_sizes, embed, pos_enc, type_W,
           type_b, gru_kernel, gru_rec, gru_b_in, gru_b_rec, att_w, att_b,
           att_v):
    i32 = jnp.int32
    et = edges[:, 0] % NT
    src = edges[:, 1]
    dst = edges[:, 2]
    padn = NEP - NE
    src_p = jnp.concatenate([src, jnp.zeros((padn,), i32)])
    dstrow = et * NNP + dst
    dst_p = jnp.concatenate([dstrow, jnp.full((padn,), TRASH, i32)])
    src_st = jnp.stack([src_p, src_p + NNP]).reshape(NC, NS, ECH, 128)
    dst_st = dst_p.reshape(NS, ECH, 128)
    dst_c = jnp.concatenate(
        [dstrow, jnp.full((NEPC - NE,), TRASH, i32)]).reshape(NC, NS, CCH, 128)

    nodes_p = jnp.concatenate([nodes, jnp.zeros((NNP - NN,), i32)])
    pp = jnp.where(node_positions != 0, node_positions, 64).astype(i32)
    pp_p = jnp.concatenate([pp, jnp.full((NNP - NN,), 64, i32)])
    idxn = jnp.stack([2 * nodes_p, 2 * nodes_p + 1]).reshape(NC, NS, 5, 128)
    idxp = jnp.stack([2 * pp_p, 2 * pp_p + 1]).reshape(NC, NS, 5, 128)

    embed_r = embed.reshape(-1, HD)
    posx = jnp.concatenate(
        [pos_enc, jnp.zeros((8, D), jnp.float32)]).reshape(-1, HD)

    graph_ids = jnp.repeat(jnp.arange(NG, dtype=i32), graph_sizes,
                           total_repeat_length=NN)
    seg = (jnp.arange(NG, dtype=i32)[:, None] == graph_ids[None, :]) \
        .astype(jnp.float32)

    bi = gru_b_in.reshape(1, 3 * D)
    br = gru_b_rec.reshape(1, 3 * D)
    ab = att_b.reshape(1, 1)

    states = _sc_embed(embed_r, posx, idxn, idxp)
    cnt = _sc_counts(dst_c)
    for step in range(2):
        a = _sc_edge(states.reshape(-1, HD), src_st, dst_st)
        states = _tc_gru(a, cnt, states, type_W, type_b, gru_kernel, gru_rec,
                         bi, br)
    return _tc_att(states, seg, att_w, ab, att_v)
